# Initial kernel scaffold; baseline (speedup 1.0000x reference)
import jax
import jax.numpy as jnp
from jax.experimental import pallas as pl


def kernel(x, edge_index, edge_type, batch, type_, emb0, emb1, emb2, emb3, emb4, emb5, W1_rel, W1_root, b1, W2_rel, W2_root, b2, g1_w, g1_b, bn_g, bn_b, g2_w, g2_b, gw, gb):
    raise NotImplementedError("write your pallas kernel here")


# speedup vs baseline: 3.2905x; 3.2905x over previous
import jax
import jax.numpy as jnp
from jax.experimental import pallas as pl

N = 50000
E = 800000
R = 8
D_RAW = 96
LAYER = 128
OUT = 64
OUT2 = 128
G = 64
BN = 2000


def _rgcn_block(h_ref, a_ref, wroot_ref, wstack_ref, b_ref, o_ref):
    acc = jnp.dot(h_ref[...], wroot_ref[...], preferred_element_type=jnp.float32)
    acc = acc + jnp.dot(a_ref[...], wstack_ref[...], preferred_element_type=jnp.float32)
    o_ref[...] = jax.nn.sigmoid(acc + b_ref[...])


def _rgcn_dense(h, a, wroot, wstack, b):
    n, d = h.shape
    rd = a.shape[1]
    l = wroot.shape[1]
    return pl.pallas_call(
        _rgcn_block,
        grid=(n // BN,),
        in_specs=[
            pl.BlockSpec((BN, d), lambda i: (i, 0)),
            pl.BlockSpec((BN, rd), lambda i: (i, 0)),
            pl.BlockSpec((d, l), lambda i: (0, 0)),
            pl.BlockSpec((rd, l), lambda i: (0, 0)),
            pl.BlockSpec((1, l), lambda i: (0, 0)),
        ],
        out_specs=pl.BlockSpec((BN, l), lambda i: (i, 0)),
        out_shape=jax.ShapeDtypeStruct((n, l), jnp.float32),
    )(h, a, wroot, wstack, b)


def _pool1_block(mu_ref, batch_ref, g1w_ref, g1b_ref, bng_ref, bnb_ref,
                 g2w_ref, g2b_ref, gate_ref, m_ref):
    i = pl.program_id(0)
    g = jnp.dot(mu_ref[...], g1w_ref[...], preferred_element_type=jnp.float32)
    g = (g + g1b_ref[...]) * bng_ref[...] + bnb_ref[...]
    g = jnp.maximum(g, 0.0)
    gate = jnp.sum(g * g2w_ref[...], axis=1, keepdims=True) + g2b_ref[...]
    gate_ref[...] = gate
    ids = jax.lax.broadcasted_iota(jnp.int32, (1, G), 1)
    onehot = batch_ref[...] == ids
    masked = jnp.where(onehot, gate, -1e30)
    local_m = jnp.max(masked, axis=0, keepdims=True)

    @pl.when(i == 0)
    def _():
        m_ref[...] = jnp.full((1, G), -1e30, jnp.float32)

    m_ref[...] = jnp.maximum(m_ref[...], local_m)


def _pool2_block(mu_ref, batch_ref, gate_ref, m_ref, s_ref, p_ref):
    i = pl.program_id(0)
    ids = jax.lax.broadcasted_iota(jnp.int32, (1, G), 1)
    onehot = (batch_ref[...] == ids).astype(jnp.float32)
    mb = jnp.sum(onehot * m_ref[...], axis=1, keepdims=True)
    e = jnp.exp(gate_ref[...] - mb)
    we = onehot * e
    local_s = jnp.sum(we, axis=0, keepdims=True)
    local_p = jax.lax.dot_general(
        we, mu_ref[...], (((0,), (0,)), ((), ())),
        preferred_element_type=jnp.float32)

    @pl.when(i == 0)
    def _():
        s_ref[...] = jnp.zeros((1, G), jnp.float32)
        p_ref[...] = jnp.zeros((G, OUT), jnp.float32)

    s_ref[...] += local_s
    p_ref[...] += local_p


def kernel(x, edge_index, edge_type, batch, type_, emb0, emb1, emb2, emb3,
           emb4, emb5, W1_rel, W1_root, b1, W2_rel, W2_root, b2,
           g1_w, g1_b, bn_g, bn_b, g2_w, g2_b, gw, gb):
    tables = [emb0, emb1, emb2, emb3, emb4, emb5]
    xi = x.astype(jnp.int32)
    x_ = jnp.concatenate([jnp.take(tables[i], xi[:, i], axis=0)
                          for i in range(6)], axis=1)

    src = edge_index[0].astype(jnp.int32)
    dst = edge_index[1].astype(jnp.int32)
    et = edge_type.astype(jnp.int32)
    key = dst * R + et
    cnt = jax.ops.segment_sum(jnp.ones((E,), jnp.float32), key,
                              num_segments=N * R)
    inv = (1.0 / jnp.clip(cnt, 1.0, None))[:, None]

    msum1 = jax.ops.segment_sum(jnp.take(x_, src, axis=0), key,
                                num_segments=N * R)
    a1 = (msum1 * inv).reshape(N, R * D_RAW)
    h0 = jnp.pad(x_, ((0, 0), (0, LAYER - D_RAW)))
    w1root = jnp.pad(W1_root, ((0, LAYER - D_RAW), (0, 0)))
    h1 = _rgcn_dense(h0, a1, w1root, W1_rel.reshape(R * D_RAW, LAYER),
                     b1.reshape(1, LAYER))

    msum2 = jax.ops.segment_sum(jnp.take(h1, src, axis=0), key,
                                num_segments=N * R)
    a2 = (msum2 * inv).reshape(N, R * LAYER)
    h2 = _rgcn_dense(h1, a2, W2_root, W2_rel.reshape(R * LAYER, OUT2),
                     b2.reshape(1, OUT2))

    mu = h2[:, :OUT]
    batch2 = batch.astype(jnp.int32).reshape(N, 1)
    scale = 1.0 / jnp.sqrt(jnp.float32(1.0 + 1e-5))
    bng = (bn_g * scale).reshape(1, OUT)

    gate, m = pl.pallas_call(
        _pool1_block,
        grid=(N // BN,),
        in_specs=[
            pl.BlockSpec((BN, OUT), lambda i: (i, 0)),
            pl.BlockSpec((BN, 1), lambda i: (i, 0)),
            pl.BlockSpec((OUT, OUT), lambda i: (0, 0)),
            pl.BlockSpec((1, OUT), lambda i: (0, 0)),
            pl.BlockSpec((1, OUT), lambda i: (0, 0)),
            pl.BlockSpec((1, OUT), lambda i: (0, 0)),
            pl.BlockSpec((1, OUT), lambda i: (0, 0)),
            pl.BlockSpec((1, 1), lambda i: (0, 0)),
        ],
        out_specs=[
            pl.BlockSpec((BN, 1), lambda i: (i, 0)),
            pl.BlockSpec((1, G), lambda i: (0, 0)),
        ],
        out_shape=[
            jax.ShapeDtypeStruct((N, 1), jnp.float32),
            jax.ShapeDtypeStruct((1, G), jnp.float32),
        ],
    )(mu, batch2, g1_w, g1_b.reshape(1, OUT), bng, bn_b.reshape(1, OUT),
      g2_w.reshape(1, OUT), g2_b.reshape(1, 1))

    s, p = pl.pallas_call(
        _pool2_block,
        grid=(N // BN,),
        in_specs=[
            pl.BlockSpec((BN, OUT), lambda i: (i, 0)),
            pl.BlockSpec((BN, 1), lambda i: (i, 0)),
            pl.BlockSpec((BN, 1), lambda i: (i, 0)),
            pl.BlockSpec((1, G), lambda i: (0, 0)),
        ],
        out_specs=[
            pl.BlockSpec((1, G), lambda i: (0, 0)),
            pl.BlockSpec((G, OUT), lambda i: (0, 0)),
        ],
        out_shape=[
            jax.ShapeDtypeStruct((1, G), jnp.float32),
            jax.ShapeDtypeStruct((G, OUT), jnp.float32),
        ],
    )(mu, batch2, gate, m)

    pooled = p / jnp.maximum(s, 1e-30).reshape(G, 1)
    return jax.nn.sigmoid(pooled @ gw + gb)


# speedup vs baseline: 3.6224x; 1.1009x over previous
import jax
import jax.numpy as jnp
from jax.experimental import pallas as pl

N = 50000
E = 800000
R = 8
D_RAW = 96
LAYER = 128
OUT = 64
OUT2 = 128
G = 64
BN = 2000


def _rgcn_block(h_ref, a_ref, wroot_ref, wstack_ref, b_ref, o_ref):
    acc = jnp.dot(h_ref[...], wroot_ref[...], preferred_element_type=jnp.float32)
    acc = acc + jnp.dot(a_ref[...], wstack_ref[...], preferred_element_type=jnp.float32)
    o_ref[...] = jax.nn.sigmoid(acc + b_ref[...])


def _rgcn_dense(h, a, wroot, wstack, b):
    n, d = h.shape
    rd = a.shape[1]
    l = wroot.shape[1]
    return pl.pallas_call(
        _rgcn_block,
        grid=(n // BN,),
        in_specs=[
            pl.BlockSpec((BN, d), lambda i: (i, 0)),
            pl.BlockSpec((BN, rd), lambda i: (i, 0)),
            pl.BlockSpec((d, l), lambda i: (0, 0)),
            pl.BlockSpec((rd, l), lambda i: (0, 0)),
            pl.BlockSpec((1, l), lambda i: (0, 0)),
        ],
        out_specs=pl.BlockSpec((BN, l), lambda i: (i, 0)),
        out_shape=jax.ShapeDtypeStruct((n, l), jnp.float32),
    )(h, a, wroot, wstack, b)


def _pool1_block(mu_ref, batch_ref, g1w_ref, g1b_ref, bng_ref, bnb_ref,
                 g2w_ref, g2b_ref, gate_ref, m_ref):
    i = pl.program_id(0)
    g = jnp.dot(mu_ref[...], g1w_ref[...], preferred_element_type=jnp.float32)
    g = (g + g1b_ref[...]) * bng_ref[...] + bnb_ref[...]
    g = jnp.maximum(g, 0.0)
    gate = jnp.sum(g * g2w_ref[...], axis=1, keepdims=True) + g2b_ref[...]
    gate_ref[...] = gate
    ids = jax.lax.broadcasted_iota(jnp.int32, (1, G), 1)
    onehot = batch_ref[...] == ids
    masked = jnp.where(onehot, gate, -1e30)
    local_m = jnp.max(masked, axis=0, keepdims=True)

    @pl.when(i == 0)
    def _():
        m_ref[...] = jnp.full((1, G), -1e30, jnp.float32)

    m_ref[...] = jnp.maximum(m_ref[...], local_m)


def _pool2_block(mu_ref, batch_ref, gate_ref, m_ref, s_ref, p_ref):
    i = pl.program_id(0)
    ids = jax.lax.broadcasted_iota(jnp.int32, (1, G), 1)
    onehot = (batch_ref[...] == ids).astype(jnp.float32)
    mb = jnp.sum(onehot * m_ref[...], axis=1, keepdims=True)
    e = jnp.exp(gate_ref[...] - mb)
    we = onehot * e
    local_s = jnp.sum(we, axis=0, keepdims=True)
    local_p = jax.lax.dot_general(
        we, mu_ref[...], (((0,), (0,)), ((), ())),
        preferred_element_type=jnp.float32)

    @pl.when(i == 0)
    def _():
        s_ref[...] = jnp.zeros((1, G), jnp.float32)
        p_ref[...] = jnp.zeros((G, OUT), jnp.float32)

    s_ref[...] += local_s
    p_ref[...] += local_p


def kernel(x, edge_index, edge_type, batch, type_, emb0, emb1, emb2, emb3,
           emb4, emb5, W1_rel, W1_root, b1, W2_rel, W2_root, b2,
           g1_w, g1_b, bn_g, bn_b, g2_w, g2_b, gw, gb):
    tables = [emb0, emb1, emb2, emb3, emb4, emb5]
    xi = x.astype(jnp.int32)
    x_ = jnp.concatenate([jnp.take(tables[i], xi[:, i], axis=0)
                          for i in range(6)], axis=1)

    src = edge_index[0].astype(jnp.int32)
    dst = edge_index[1].astype(jnp.int32)
    et = edge_type.astype(jnp.int32)
    key = dst * R + et

    feat1 = jnp.concatenate(
        [jnp.take(x_, src, axis=0), jnp.ones((E, 1), jnp.float32)], axis=1)
    msum1 = jax.ops.segment_sum(feat1, key, num_segments=N * R)
    cnt = msum1[:, D_RAW]
    inv = (1.0 / jnp.clip(cnt, 1.0, None))[:, None]
    a1 = (msum1[:, :D_RAW] * inv).reshape(N, R * D_RAW)
    h0 = jnp.pad(x_, ((0, 0), (0, LAYER - D_RAW)))
    w1root = jnp.pad(W1_root, ((0, LAYER - D_RAW), (0, 0)))
    h1 = _rgcn_dense(h0, a1, w1root, W1_rel.reshape(R * D_RAW, LAYER),
                     b1.reshape(1, LAYER))

    msum2 = jax.ops.segment_sum(jnp.take(h1, src, axis=0), key,
                                num_segments=N * R)
    a2 = (msum2 * inv).reshape(N, R * LAYER)
    h2 = _rgcn_dense(h1, a2, W2_root, W2_rel.reshape(R * LAYER, OUT2),
                     b2.reshape(1, OUT2))

    mu = h2[:, :OUT]
    batch2 = batch.astype(jnp.int32).reshape(N, 1)
    scale = 1.0 / jnp.sqrt(jnp.float32(1.0 + 1e-5))
    bng = (bn_g * scale).reshape(1, OUT)

    gate, m = pl.pallas_call(
        _pool1_block,
        grid=(N // BN,),
        in_specs=[
            pl.BlockSpec((BN, OUT), lambda i: (i, 0)),
            pl.BlockSpec((BN, 1), lambda i: (i, 0)),
            pl.BlockSpec((OUT, OUT), lambda i: (0, 0)),
            pl.BlockSpec((1, OUT), lambda i: (0, 0)),
            pl.BlockSpec((1, OUT), lambda i: (0, 0)),
            pl.BlockSpec((1, OUT), lambda i: (0, 0)),
            pl.BlockSpec((1, OUT), lambda i: (0, 0)),
            pl.BlockSpec((1, 1), lambda i: (0, 0)),
        ],
        out_specs=[
            pl.BlockSpec((BN, 1), lambda i: (i, 0)),
            pl.BlockSpec((1, G), lambda i: (0, 0)),
        ],
        out_shape=[
            jax.ShapeDtypeStruct((N, 1), jnp.float32),
            jax.ShapeDtypeStruct((1, G), jnp.float32),
        ],
    )(mu, batch2, g1_w, g1_b.reshape(1, OUT), bng, bn_b.reshape(1, OUT),
      g2_w.reshape(1, OUT), g2_b.reshape(1, 1))

    s, p = pl.pallas_call(
        _pool2_block,
        grid=(N // BN,),
        in_specs=[
            pl.BlockSpec((BN, OUT), lambda i: (i, 0)),
            pl.BlockSpec((BN, 1), lambda i: (i, 0)),
            pl.BlockSpec((BN, 1), lambda i: (i, 0)),
            pl.BlockSpec((1, G), lambda i: (0, 0)),
        ],
        out_specs=[
            pl.BlockSpec((1, G), lambda i: (0, 0)),
            pl.BlockSpec((G, OUT), lambda i: (0, 0)),
        ],
        out_shape=[
            jax.ShapeDtypeStruct((1, G), jnp.float32),
            jax.ShapeDtypeStruct((G, OUT), jnp.float32),
        ],
    )(mu, batch2, gate, m)

    pooled = p / jnp.maximum(s, 1e-30).reshape(G, 1)
    return jax.nn.sigmoid(pooled @ gw + gb)
